# Initial kernel scaffold; baseline (speedup 1.0000x reference)
import jax
import jax.numpy as jnp
from jax.experimental import pallas as pl


def kernel(tensor, indices):
    raise NotImplementedError("write your pallas kernel here")


# speedup vs baseline: 1.3649x; 1.3649x over previous
import functools

import jax
import jax.numpy as jnp
from jax import lax
from jax.experimental import pallas as pl
from jax.experimental.pallas import tpu as pltpu
from jax.experimental.pallas import tpu_sc as plsc

B, R, D, K = 4096, 100, 128, 64
N = B * R
NW = 32
RW = N // NW
C = 128
NCHUNK = RW // C


def _sc_gather(tensor_flat, idx_flat):
    mesh = plsc.VectorSubcoreMesh(core_axis_name="c", subcore_axis_name="s")

    @functools.partial(
        pl.kernel,
        mesh=mesh,
        out_type=jax.ShapeDtypeStruct((N * K,), jnp.float32),
        scratch_types=[
            pltpu.VMEM((C * D,), jnp.float32),
            pltpu.VMEM((C * K,), jnp.int32),
            pltpu.VMEM((C * K,), jnp.float32),
        ],
        compiler_params=pltpu.CompilerParams(needs_layout_passes=False),
    )
    def k(t_hbm, i_hbm, o_hbm, rows_v, idx_v, out_v):
        wid = lax.axis_index("s") * 2 + lax.axis_index("c")
        row0 = wid * RW

        def chunk_body(ci, _):
            base = row0 + ci * C
            pltpu.sync_copy(t_hbm.at[pl.ds(base * D, C * D)], rows_v)
            pltpu.sync_copy(i_hbm.at[pl.ds(base * K, C * K)], idx_v)

            def row_body(r, _):
                rbase = r * D
                for j in range(K // 16):
                    col = idx_v[pl.ds(r * K + j * 16, 16)]
                    vals = plsc.load_gather(rows_v, [col + rbase])
                    out_v[pl.ds(r * K + j * 16, 16)] = vals
                return _

            lax.fori_loop(0, C, row_body, 0)
            pltpu.sync_copy(out_v, o_hbm.at[pl.ds(base * K, C * K)])
            return _

        lax.fori_loop(0, NCHUNK, chunk_body, 0)

    return k(tensor_flat, idx_flat)


def kernel(tensor, indices):
    out = _sc_gather(tensor.reshape(-1), indices.reshape(-1))
    return out.reshape(B, R, K)


# speedup vs baseline: 1.6509x; 1.2095x over previous
import functools

import jax
import jax.numpy as jnp
from jax import lax
from jax.experimental import pallas as pl
from jax.experimental.pallas import tpu as pltpu
from jax.experimental.pallas import tpu_sc as plsc

B, R, D, K = 4096, 100, 128, 64
N = B * R
NW = 32
RW = N // NW
C = 128
NCHUNK = RW // C
G2 = NCHUNK // 2


def _sc_gather(tensor_flat, idx_flat):
    mesh = plsc.VectorSubcoreMesh(core_axis_name="c", subcore_axis_name="s")

    @functools.partial(
        pl.kernel,
        mesh=mesh,
        out_type=jax.ShapeDtypeStruct((N * K,), jnp.float32),
        scratch_types=[
            pltpu.VMEM((C * D,), jnp.float32),
            pltpu.VMEM((C * D,), jnp.float32),
            pltpu.VMEM((C * K,), jnp.int32),
            pltpu.VMEM((C * K,), jnp.int32),
            pltpu.VMEM((C * K,), jnp.float32),
            pltpu.VMEM((C * K,), jnp.float32),
            pltpu.SemaphoreType.DMA,
            pltpu.SemaphoreType.DMA,
            pltpu.SemaphoreType.DMA,
            pltpu.SemaphoreType.DMA,
        ],
        compiler_params=pltpu.CompilerParams(needs_layout_passes=False),
    )
    def k(t_hbm, i_hbm, o_hbm, rows0, rows1, idx0, idx1, out0, out1,
          si0, si1, so0, so1):
        wid = lax.axis_index("s") * 2 + lax.axis_index("c")
        row0 = wid * RW
        rows, idxv, outv = (rows0, rows1), (idx0, idx1), (out0, out1)
        sin, sout = (si0, si1), (so0, so1)

        def start_load(ci, b):
            base = row0 + ci * C
            pltpu.make_async_copy(
                t_hbm.at[pl.ds(base * D, C * D)], rows[b], sin[b]).start()
            pltpu.make_async_copy(
                i_hbm.at[pl.ds(base * K, C * K)], idxv[b], sin[b]).start()

        def wait_load(b):
            pltpu.make_async_copy(
                t_hbm.at[pl.ds(row0 * D, C * D)], rows[b], sin[b]).wait()
            pltpu.make_async_copy(
                i_hbm.at[pl.ds(row0 * K, C * K)], idxv[b], sin[b]).wait()

        def start_store(ci, b):
            base = row0 + ci * C
            pltpu.make_async_copy(
                outv[b], o_hbm.at[pl.ds(base * K, C * K)], sout[b]).start()

        def wait_store(b):
            pltpu.make_async_copy(
                outv[b], o_hbm.at[pl.ds(row0 * K, C * K)], sout[b]).wait()

        def compute(b):
            @pl.loop(0, C, unroll=4)
            def row_body(r):
                rbase = r * D
                for j in range(K // 16):
                    col = idxv[b][pl.ds(r * K + j * 16, 16)]
                    outv[b][pl.ds(r * K + j * 16, 16)] = plsc.load_gather(
                        rows[b], [col + rbase])

        start_load(0, 0)
        start_load(1, 1)
        for b in (0, 1):
            wait_load(b)
            compute(b)
            start_store(b, b)
            start_load(b + 2, b)

        def body(g, carry):
            for b in (0, 1):
                ci = 2 * g + b
                wait_load(b)
                wait_store(b)
                compute(b)
                start_store(ci, b)
                start_load(ci + 2, b)
            return carry

        lax.fori_loop(1, G2 - 1, body, 0)

        for b in (0, 1):
            wait_load(b)
            wait_store(b)
            compute(b)
            start_store(2 * (G2 - 1) + b, b)
        for b in (0, 1):
            wait_store(b)

    return k(tensor_flat, idx_flat)


def kernel(tensor, indices):
    out = _sc_gather(tensor.reshape(-1), indices.reshape(-1))
    return out.reshape(B, R, K)


# speedup vs baseline: 2.0578x; 1.2465x over previous
import functools

import jax
import jax.numpy as jnp
from jax import lax
from jax.experimental import pallas as pl
from jax.experimental.pallas import tpu as pltpu
from jax.experimental.pallas import tpu_sc as plsc

B, R, D, K = 4096, 100, 128, 64
NW = 32
BW = B // NW
G2 = BW // 2


def _sc_gather(tensor, indices):
    mesh = plsc.VectorSubcoreMesh(core_axis_name="c", subcore_axis_name="s")

    @functools.partial(
        pl.kernel,
        mesh=mesh,
        out_type=jax.ShapeDtypeStruct((B, R, K), jnp.float32),
        scratch_types=[
            pltpu.VMEM((R, D), jnp.float32),
            pltpu.VMEM((R, D), jnp.float32),
            pltpu.VMEM((R, K), jnp.int32),
            pltpu.VMEM((R, K), jnp.int32),
            pltpu.VMEM((R, K), jnp.float32),
            pltpu.VMEM((R, K), jnp.float32),
            pltpu.SemaphoreType.DMA,
            pltpu.SemaphoreType.DMA,
            pltpu.SemaphoreType.DMA,
            pltpu.SemaphoreType.DMA,
        ],
        compiler_params=pltpu.CompilerParams(needs_layout_passes=False),
    )
    def k(t_hbm, i_hbm, o_hbm, rows0, rows1, idx0, idx1, out0, out1,
          si0, si1, so0, so1):
        wid = lax.axis_index("s") * 2 + lax.axis_index("c")
        b0 = wid * BW
        rows, idxv, outv = (rows0, rows1), (idx0, idx1), (out0, out1)
        sin, sout = (si0, si1), (so0, so1)

        def start_load(ci, b):
            pltpu.make_async_copy(t_hbm.at[b0 + ci], rows[b], sin[b]).start()
            pltpu.make_async_copy(i_hbm.at[b0 + ci], idxv[b], sin[b]).start()

        def wait_load(b):
            pltpu.make_async_copy(t_hbm.at[b0], rows[b], sin[b]).wait()
            pltpu.make_async_copy(i_hbm.at[b0], idxv[b], sin[b]).wait()

        def start_store(ci, b):
            pltpu.make_async_copy(outv[b], o_hbm.at[b0 + ci], sout[b]).start()

        def wait_store(b):
            pltpu.make_async_copy(outv[b], o_hbm.at[b0], sout[b]).wait()

        def compute(b):
            @pl.loop(0, R, unroll=4)
            def row_body(r):
                rvec = jnp.full((16,), r, jnp.int32)
                for j in range(K // 16):
                    col = idxv[b][r, pl.ds(j * 16, 16)]
                    outv[b][r, pl.ds(j * 16, 16)] = plsc.load_gather(
                        rows[b], [rvec, col])

        start_load(0, 0)
        start_load(1, 1)
        for b in (0, 1):
            wait_load(b)
            compute(b)
            start_store(b, b)
            start_load(b + 2, b)

        def body(g, carry):
            for b in (0, 1):
                ci = 2 * g + b
                wait_load(b)
                wait_store(b)
                compute(b)
                start_store(ci, b)
                start_load(ci + 2, b)
            return carry

        lax.fori_loop(1, G2 - 1, body, 0)

        for b in (0, 1):
            wait_load(b)
            wait_store(b)
            compute(b)
            start_store(2 * (G2 - 1) + b, b)
        for b in (0, 1):
            wait_store(b)

    return k(tensor, indices)


def kernel(tensor, indices):
    return _sc_gather(tensor, indices)


# speedup vs baseline: 3.1036x; 1.5082x over previous
import functools

import jax
import jax.numpy as jnp
from jax import lax
from jax.experimental import pallas as pl
from jax.experimental.pallas import tpu as pltpu
from jax.experimental.pallas import tpu_sc as plsc

B, R, D, K = 4096, 100, 128, 64
NW = 32
BW = B // NW
G2 = BW // 2


def _sc_gather(tensor, indices):
    mesh = plsc.VectorSubcoreMesh(core_axis_name="c", subcore_axis_name="s")

    @functools.partial(
        pl.kernel,
        mesh=mesh,
        out_type=jax.ShapeDtypeStruct((B, R, K), jnp.float32),
        scratch_types=[
            pltpu.VMEM((R, D), jnp.float32),
            pltpu.VMEM((R, D), jnp.float32),
            pltpu.VMEM((R, K), jnp.int32),
            pltpu.VMEM((R, K), jnp.int32),
            pltpu.VMEM((R, K), jnp.float32),
            pltpu.VMEM((R, K), jnp.float32),
            pltpu.SemaphoreType.DMA,
            pltpu.SemaphoreType.DMA,
            pltpu.SemaphoreType.DMA,
            pltpu.SemaphoreType.DMA,
        ],
        compiler_params=pltpu.CompilerParams(needs_layout_passes=False),
    )
    def k(t_hbm, i_hbm, o_hbm, rows0, rows1, idx0, idx1, out0, out1,
          si0, si1, so0, so1):
        wid = lax.axis_index("s") * 2 + lax.axis_index("c")
        b0 = wid * BW
        rows, idxv, outv = (rows0, rows1), (idx0, idx1), (out0, out1)
        sin, sout = (si0, si1), (so0, so1)

        def start_load(ci, b):
            pltpu.make_async_copy(t_hbm.at[b0 + ci], rows[b], sin[b]).start()
            pltpu.make_async_copy(i_hbm.at[b0 + ci], idxv[b], sin[b]).start()

        def wait_load(b):
            pltpu.make_async_copy(t_hbm.at[b0], rows[b], sin[b]).wait()
            pltpu.make_async_copy(i_hbm.at[b0], idxv[b], sin[b]).wait()

        def start_store(ci, b):
            pltpu.make_async_copy(outv[b], o_hbm.at[b0 + ci], sout[b]).start()

        def wait_store(b):
            pltpu.make_async_copy(outv[b], o_hbm.at[b0], sout[b]).wait()

        def compute(b):
            @plsc.parallel_loop(0, R, 1, unroll=4)
            def row_body(r):
                rvec = jnp.full((16,), r, jnp.int32)
                for j in range(K // 16):
                    col = idxv[b][r, pl.ds(j * 16, 16)]
                    outv[b][r, pl.ds(j * 16, 16)] = plsc.load_gather(
                        rows[b], [rvec, col])

        start_load(0, 0)
        start_load(1, 1)
        for b in (0, 1):
            wait_load(b)
            compute(b)
            start_store(b, b)
            start_load(b + 2, b)

        def body(g, carry):
            for b in (0, 1):
                ci = 2 * g + b
                wait_load(b)
                wait_store(b)
                compute(b)
                start_store(ci, b)
                start_load(ci + 2, b)
            return carry

        lax.fori_loop(1, G2 - 1, body, 0)

        for b in (0, 1):
            wait_load(b)
            wait_store(b)
            compute(b)
            start_store(2 * (G2 - 1) + b, b)
        for b in (0, 1):
            wait_store(b)

    return k(tensor, indices)


def kernel(tensor, indices):
    return _sc_gather(tensor, indices)
